# baseline (device time: 18379 ns/iter reference)
import jax
import jax.numpy as jnp
from jax import lax
from jax.experimental import pallas as pl
from jax.experimental.pallas import tpu as pltpu

N_DEV = 4
B = 2
SQ = 128
SKV = 512
HQ = 4
DH = 64
D_MODEL = 512
D_QK = HQ * DH
BLK = 64


def kernel(x, Wq, K_ext, V_ext, Wo):
    bf16 = jnp.bfloat16
    k2 = K_ext.reshape(B, SQ, D_QK)
    v2 = V_ext.reshape(B, SQ, D_QK)

    def body(x_ref, wq_ref, k_ref, v_ref, wo_ref, out_ref,
             comm, send_sems, recv_sems):
        mp = lax.axis_index("i")
        left = (mp - 1) % N_DEV
        right = (mp + 1) % N_DEV

        barrier = pltpu.get_barrier_semaphore()
        for nbr in (left, right):
            pl.semaphore_signal(barrier, inc=1, device_id=(nbr,),
                                device_id_type=pl.DeviceIdType.MESH)
        pl.semaphore_wait(barrier, 2)

        MINE, A, BS, C = 0, 1, 2, 3
        comm[MINE, 0:B] = k_ref[...].astype(bf16)
        comm[MINE, B:2 * B] = v_ref[...].astype(bf16)

        def make_rdma(src_slot, dst_slot, sem_idx, dev):
            return pltpu.make_async_remote_copy(
                src_ref=comm.at[src_slot],
                dst_ref=comm.at[dst_slot],
                send_sem=send_sems.at[sem_idx],
                recv_sem=recv_sems.at[sem_idx],
                device_id=(dev,),
                device_id_type=pl.DeviceIdType.MESH,
            )

        r_to_right = make_rdma(MINE, A, 0, right)
        r_to_left = make_rdma(MINE, BS, 1, left)
        r_to_right.start()
        r_to_left.start()

        wq = wq_ref[...].astype(bf16)
        q = []
        for b in range(B):
            qb = lax.dot_general(x_ref[b].astype(bf16), wq,
                                 (((1,), (0,)), ((), ())),
                                 preferred_element_type=jnp.float32)
            q.append(qb.astype(bf16))

        slot_of_s = {0: MINE, 1: BS, 2: C, 3: A}

        row = lax.broadcasted_iota(jnp.int32, (SQ, SQ), 0) + mp * SQ
        colr = lax.broadcasted_iota(jnp.int32, (SQ, SQ), 1)
        qblk = row // BLK
        masks = []
        for s in range(N_DEV):
            col = colr + ((mp + s) % N_DEV) * SQ
            kblk = col // BLK
            masks.append((qblk == kblk) | (kblk == 0)
                         | ((qblk + kblk) % 3 == 0))

        scores = [[[None] * N_DEV for _ in range(HQ)] for _ in range(B)]

        def do_scores(s):
            slot = slot_of_s[s]
            for b in range(B):
                kb = comm[slot, b]
                for h in range(HQ):
                    qh = q[b][:, h * DH:(h + 1) * DH]
                    kh = kb[:, h * DH:(h + 1) * DH]
                    sc = lax.dot_general(qh, kh, (((1,), (1,)), ((), ())),
                                         preferred_element_type=jnp.float32)
                    scores[b][h][s] = jnp.where(masks[s], sc * 0.125, -1e9)

        do_scores(0)

        r_to_right.wait_recv()
        r_fwd = make_rdma(A, C, 2, right)
        r_fwd.start()
        do_scores(3)

        r_to_left.wait_recv()
        do_scores(1)

        r_fwd.wait_recv()
        do_scores(2)

        wo = wo_ref[...].astype(bf16)
        for b in range(B):
            ctx_heads = []
            for h in range(HQ):
                sc = jnp.concatenate(scores[b][h], axis=1)
                m = jnp.max(sc, axis=-1, keepdims=True)
                e = jnp.exp(sc - m)
                w = (e / jnp.sum(e, axis=-1, keepdims=True)).astype(bf16)
                acc = None
                for s in range(N_DEV):
                    vb = comm[slot_of_s[s], B + b]
                    vh = vb[:, h * DH:(h + 1) * DH]
                    ws = w[:, s * SQ:(s + 1) * SQ]
                    p = lax.dot_general(ws, vh, (((1,), (0,)), ((), ())),
                                        preferred_element_type=jnp.float32)
                    acc = p if acc is None else acc + p
                ctx_heads.append(acc)
            ctx = jnp.concatenate(ctx_heads, axis=1).astype(bf16)
            out_ref[b] = lax.dot_general(ctx, wo, (((1,), (0,)), ((), ())),
                                         preferred_element_type=jnp.float32)

        r_to_right.wait_send()
        r_to_left.wait_send()
        r_fwd.wait_send()

    return pl.pallas_call(
        body,
        out_shape=jax.ShapeDtypeStruct((B, SQ, D_MODEL), jnp.float32),
        in_specs=[pl.BlockSpec(memory_space=pltpu.VMEM)] * 5,
        out_specs=pl.BlockSpec(memory_space=pltpu.VMEM),
        scratch_shapes=[
            pltpu.VMEM((N_DEV, 2 * B, SQ, D_QK), bf16),
            pltpu.SemaphoreType.DMA((3,)),
            pltpu.SemaphoreType.DMA((3,)),
        ],
        compiler_params=pltpu.CompilerParams(collective_id=0),
    )(x, Wq, k2, v2, Wo)


# device time: 16921 ns/iter; 1.0862x vs baseline; 1.0862x over previous
import jax
import jax.numpy as jnp
from jax import lax
from jax.experimental import pallas as pl
from jax.experimental.pallas import tpu as pltpu

N_DEV = 4
B = 2
SQ = 128
SKV = 512
HQ = 4
DH = 64
D_MODEL = 512
D_QK = HQ * DH
BLK = 64
HALF = SQ // 2


def kernel(x, Wq, K_ext, V_ext, Wo):
    bf16 = jnp.bfloat16
    k2 = K_ext.reshape(B, SQ, D_QK)
    v2 = V_ext.reshape(B, SQ, D_QK)

    def body(x_ref, wq_ref, k_ref, v_ref, wo_ref, out_ref,
             comm, send_sems, recv_sems):
        mp = lax.axis_index("i")
        left = (mp - 1) % N_DEV
        right = (mp + 1) % N_DEV

        MINE, A, BS, C = 0, 1, 2, 3
        comm[MINE, 0:B] = k_ref[...].astype(bf16)
        comm[MINE, B:2 * B] = v_ref[...].astype(bf16)

        barrier = pltpu.get_barrier_semaphore()
        for nbr in (left, right):
            pl.semaphore_signal(barrier, inc=1, device_id=(nbr,),
                                device_id_type=pl.DeviceIdType.MESH)
        pl.semaphore_wait(barrier, 2)

        LO = pl.ds(0, HALF)
        HI = pl.ds(HALF, HALF)

        def make_rdma(src_slot, dst_slot, rows, sem_idx, dev):
            return pltpu.make_async_remote_copy(
                src_ref=comm.at[src_slot, :, rows],
                dst_ref=comm.at[dst_slot, :, rows],
                send_sem=send_sems.at[sem_idx],
                recv_sem=recv_sems.at[sem_idx],
                device_id=(dev,),
                device_id_type=pl.DeviceIdType.MESH,
            )

        r0 = make_rdma(MINE, A, LO, 0, right)
        r1 = make_rdma(MINE, BS, HI, 1, left)
        r2 = make_rdma(MINE, A, HI, 2, right)
        r3 = make_rdma(MINE, BS, LO, 3, left)
        r0.start()
        r1.start()
        r2.start()
        r3.start()

        wq = wq_ref[...].astype(bf16)
        q = []
        for b in range(B):
            qb = lax.dot_general(x_ref[b].astype(bf16), wq,
                                 (((1,), (0,)), ((), ())),
                                 preferred_element_type=jnp.float32)
            q.append(qb.astype(bf16))

        slot_of_s = {0: MINE, 1: BS, 2: C, 3: A}

        row = lax.broadcasted_iota(jnp.int32, (SQ, SQ), 0) + mp * SQ
        colr = lax.broadcasted_iota(jnp.int32, (SQ, SQ), 1)
        qblk = row // BLK
        masks = []
        for s in range(N_DEV):
            col = colr + ((mp + s) % N_DEV) * SQ
            kblk = col // BLK
            masks.append((qblk == kblk) | (kblk == 0)
                         | ((qblk + kblk) % 3 == 0))

        def block_scores(s, b, h):
            kb = comm[slot_of_s[s], b]
            qh = q[b][:, h * DH:(h + 1) * DH]
            kh = kb[:, h * DH:(h + 1) * DH]
            sc = lax.dot_general(qh, kh, (((1,), (1,)), ((), ())),
                                 preferred_element_type=jnp.float32)
            return jnp.where(masks[s], sc * 0.125, -1e9)

        def block_pv(s, b, h, e_bf):
            vb = comm[slot_of_s[s], B + b]
            vh = vb[:, h * DH:(h + 1) * DH]
            return lax.dot_general(e_bf, vh, (((1,), (0,)), ((), ())),
                                   preferred_element_type=jnp.float32)

        sc0 = [[block_scores(0, b, h) for h in range(HQ)] for b in range(B)]

        r0.wait_recv()
        f_r = make_rdma(A, C, LO, 4, right)
        f_r.start()
        r1.wait_recv()
        f_l = make_rdma(BS, C, HI, 5, left)
        f_l.start()

        r2.wait_recv()
        r3.wait_recv()

        m_p = [[None] * HQ for _ in range(B)]
        d_p = [[None] * HQ for _ in range(B)]
        ctx_p = [[None] * HQ for _ in range(B)]
        for b in range(B):
            for h in range(HQ):
                s3 = block_scores(3, b, h)
                s1 = block_scores(1, b, h)
                m = jnp.maximum(
                    jnp.max(sc0[b][h], axis=-1, keepdims=True),
                    jnp.maximum(jnp.max(s3, axis=-1, keepdims=True),
                                jnp.max(s1, axis=-1, keepdims=True)))
                acc = None
                den = None
                for s, sc in ((0, sc0[b][h]), (3, s3), (1, s1)):
                    e = jnp.exp(sc - m)
                    den = e.sum(-1, keepdims=True) if den is None \
                        else den + e.sum(-1, keepdims=True)
                    p = block_pv(s, b, h, e.astype(bf16))
                    acc = p if acc is None else acc + p
                m_p[b][h] = m
                d_p[b][h] = den
                ctx_p[b][h] = acc

        f_r.wait_recv()
        f_l.wait_recv()

        wo = wo_ref[...].astype(bf16)
        for b in range(B):
            ctx_heads = []
            for h in range(HQ):
                sc = block_scores(2, b, h)
                mc = jnp.max(sc, axis=-1, keepdims=True)
                m = jnp.maximum(m_p[b][h], mc)
                scale = jnp.exp(m_p[b][h] - m)
                e = jnp.exp(sc - m)
                den = d_p[b][h] * scale + e.sum(-1, keepdims=True)
                ctx = ctx_p[b][h] * scale + block_pv(2, b, h, e.astype(bf16))
                ctx_heads.append(ctx / den)
            ctx = jnp.concatenate(ctx_heads, axis=1).astype(bf16)
            out_ref[b] = lax.dot_general(ctx, wo, (((1,), (0,)), ((), ())),
                                         preferred_element_type=jnp.float32)

        for r in (r0, r1, r2, r3, f_r, f_l):
            r.wait_send()

    return pl.pallas_call(
        body,
        out_shape=jax.ShapeDtypeStruct((B, SQ, D_MODEL), jnp.float32),
        in_specs=[pl.BlockSpec(memory_space=pltpu.VMEM)] * 5,
        out_specs=pl.BlockSpec(memory_space=pltpu.VMEM),
        scratch_shapes=[
            pltpu.VMEM((N_DEV, 2 * B, SQ, D_QK), bf16),
            pltpu.SemaphoreType.DMA((6,)),
            pltpu.SemaphoreType.DMA((6,)),
        ],
        compiler_params=pltpu.CompilerParams(collective_id=0),
    )(x, Wq, k2, v2, Wo)


# device time: 16366 ns/iter; 1.1230x vs baseline; 1.0339x over previous
import jax
import jax.numpy as jnp
from jax import lax
from jax.experimental import pallas as pl
from jax.experimental.pallas import tpu as pltpu

N_DEV = 4
B = 2
SQ = 128
SKV = 512
HQ = 4
DH = 64
D_MODEL = 512
D_QK = HQ * DH
BLK = 64
HALF = SQ // 2


def kernel(x, Wq, K_ext, V_ext, Wo):
    bf16 = jnp.bfloat16

    def body(x_ref, wq_ref, k_ref, v_ref, wo_ref, out_ref,
             comm, send_sems, recv_sems):
        mp = lax.axis_index("i")
        left = (mp - 1) % N_DEV
        right = (mp + 1) % N_DEV

        MINE, A, BS, C = 0, 1, 2, 3
        comm[MINE, 0:B] = k_ref[...].reshape(B, SQ, D_QK).astype(bf16)
        comm[MINE, B:2 * B] = v_ref[...].reshape(B, SQ, D_QK).astype(bf16)

        barrier = pltpu.get_barrier_semaphore()
        for nbr in (left, right):
            pl.semaphore_signal(barrier, inc=1, device_id=(nbr,),
                                device_id_type=pl.DeviceIdType.MESH)
        pl.semaphore_wait(barrier, 2)

        LO = pl.ds(0, HALF)
        HI = pl.ds(HALF, HALF)

        def make_rdma(src_slot, dst_slot, rows, sem_idx, dev):
            return pltpu.make_async_remote_copy(
                src_ref=comm.at[src_slot, :, rows],
                dst_ref=comm.at[dst_slot, :, rows],
                send_sem=send_sems.at[sem_idx],
                recv_sem=recv_sems.at[sem_idx],
                device_id=(dev,),
                device_id_type=pl.DeviceIdType.MESH,
            )

        r0 = make_rdma(MINE, A, LO, 0, right)
        r1 = make_rdma(MINE, BS, HI, 1, left)
        r2 = make_rdma(MINE, A, HI, 2, right)
        r3 = make_rdma(MINE, BS, LO, 3, left)
        r0.start()
        r1.start()
        r2.start()
        r3.start()

        wq = wq_ref[...].astype(bf16)
        q = []
        for b in range(B):
            qb = lax.dot_general(x_ref[b].astype(bf16), wq,
                                 (((1,), (0,)), ((), ())),
                                 preferred_element_type=jnp.float32)
            q.append(qb.astype(bf16))

        slot_of_s = {0: MINE, 1: BS, 2: C, 3: A}

        row = lax.broadcasted_iota(jnp.int32, (SQ, SQ), 0) + mp * SQ
        colr = lax.broadcasted_iota(jnp.int32, (SQ, SQ), 1)
        qblk = row // BLK
        masks = []
        for s in range(N_DEV):
            col = colr + ((mp + s) % N_DEV) * SQ
            kblk = col // BLK
            masks.append((qblk == kblk) | (kblk == 0)
                         | ((qblk + kblk) % 3 == 0))

        def block_scores(s, b, h):
            kb = comm[slot_of_s[s], b]
            qh = q[b][:, h * DH:(h + 1) * DH]
            kh = kb[:, h * DH:(h + 1) * DH]
            sc = lax.dot_general(qh, kh, (((1,), (1,)), ((), ())),
                                 preferred_element_type=jnp.float32)
            return jnp.where(masks[s], sc * 0.125, -1e9)

        def block_pv(s, b, h, e_bf):
            vb = comm[slot_of_s[s], B + b]
            vh = vb[:, h * DH:(h + 1) * DH]
            return lax.dot_general(e_bf, vh, (((1,), (0,)), ((), ())),
                                   preferred_element_type=jnp.float32)

        sc0 = [[block_scores(0, b, h) for h in range(HQ)] for b in range(B)]

        r0.wait_recv()
        f_r = make_rdma(A, C, LO, 4, right)
        f_r.start()
        r1.wait_recv()
        f_l = make_rdma(BS, C, HI, 5, left)
        f_l.start()

        r2.wait_recv()
        r3.wait_recv()

        m_p = [[None] * HQ for _ in range(B)]
        d_p = [[None] * HQ for _ in range(B)]
        ctx_p = [[None] * HQ for _ in range(B)]
        for b in range(B):
            for h in range(HQ):
                s3 = block_scores(3, b, h)
                s1 = block_scores(1, b, h)
                m = jnp.maximum(
                    jnp.max(sc0[b][h], axis=-1, keepdims=True),
                    jnp.maximum(jnp.max(s3, axis=-1, keepdims=True),
                                jnp.max(s1, axis=-1, keepdims=True)))
                acc = None
                den = None
                for s, sc in ((0, sc0[b][h]), (3, s3), (1, s1)):
                    e = jnp.exp(sc - m)
                    den = e.sum(-1, keepdims=True) if den is None \
                        else den + e.sum(-1, keepdims=True)
                    p = block_pv(s, b, h, e.astype(bf16))
                    acc = p if acc is None else acc + p
                m_p[b][h] = m
                d_p[b][h] = den
                ctx_p[b][h] = acc

        f_r.wait_recv()
        f_l.wait_recv()

        wo = wo_ref[...].astype(bf16)
        for b in range(B):
            ctx_heads = []
            for h in range(HQ):
                sc = block_scores(2, b, h)
                mc = jnp.max(sc, axis=-1, keepdims=True)
                m = jnp.maximum(m_p[b][h], mc)
                scale = jnp.exp(m_p[b][h] - m)
                e = jnp.exp(sc - m)
                den = d_p[b][h] * scale + e.sum(-1, keepdims=True)
                ctx = ctx_p[b][h] * scale + block_pv(2, b, h, e.astype(bf16))
                ctx_heads.append(ctx / den)
            ctx = jnp.concatenate(ctx_heads, axis=1).astype(bf16)
            out_ref[b] = lax.dot_general(ctx, wo, (((1,), (0,)), ((), ())),
                                         preferred_element_type=jnp.float32)

        for r in (r0, r1, r2, r3, f_r, f_l):
            r.wait_send()

    return pl.pallas_call(
        body,
        out_shape=jax.ShapeDtypeStruct((B, SQ, D_MODEL), jnp.float32),
        in_specs=[pl.BlockSpec(memory_space=pltpu.VMEM)] * 5,
        out_specs=pl.BlockSpec(memory_space=pltpu.VMEM),
        scratch_shapes=[
            pltpu.VMEM((N_DEV, 2 * B, SQ, D_QK), bf16),
            pltpu.SemaphoreType.DMA((6,)),
            pltpu.SemaphoreType.DMA((6,)),
        ],
        compiler_params=pltpu.CompilerParams(collective_id=0),
    )(x, Wq, K_ext, V_ext, Wo)
